# Initial kernel scaffold; baseline (speedup 1.0000x reference)
import jax
import jax.numpy as jnp
from jax.experimental import pallas as pl


def kernel(x2, edge_index2, batch, lin0_W, lin0_b, cheb_W0, cheb_W1, cheb_b):
    raise NotImplementedError("write your pallas kernel here")


# speedup vs baseline: 20.0304x; 20.0304x over previous
import functools
import jax
import jax.numpy as jnp
from jax import lax
from jax.experimental import pallas as pl
from jax.experimental.pallas import tpu as pltpu
from jax.experimental.pallas import tpu_sc as plsc

N = 10000
E = 320000
F = 128
G = 64

NC = 2
NS = 16
NW = NC * NS
EPT = E // NW
CH = 100
NCH = EPT // CH
NPAD = 10240
LW = 16

_mesh = plsc.VectorSubcoreMesh(core_axis_name="c", subcore_axis_name="s")


@functools.partial(
    pl.kernel,
    out_type=jax.ShapeDtypeStruct((NC, NPAD, F), jnp.float32),
    mesh=_mesh,
    scratch_types=[
        pltpu.VMEM((NCH, CH), jnp.int32),
        pltpu.VMEM((CH, F), jnp.float32),
        pltpu.VMEM((8, F), jnp.float32),
        pltpu.VMEM_SHARED((NPAD, F), jnp.float32),
    ],
)
def _deg_kernel(edges_hbm, out_hbm, idx_v, ones_v, zb_v, acc_sh):
    c = lax.axis_index("c")
    s = lax.axis_index("s")
    w = s * NC + c

    def fill(q, _):
        zb_v[q // 8, pl.ds((q % 8) * 16, 16)] = jnp.zeros((16,), jnp.float32)
        return 0
    lax.fori_loop(0, 64, fill, 0)

    def fill1(q, _):
        ones_v[q // 8, pl.ds((q % 8) * 16, 16)] = jnp.ones((16,), jnp.float32)
        return 0
    lax.fori_loop(0, CH * 8, fill1, 0)

    def zloop(t, _):
        pltpu.sync_copy(zb_v, acc_sh.at[pl.ds(s * 640 + t * 8, 8)])
        return 0
    lax.fori_loop(0, 80, zloop, 0)
    plsc.subcore_barrier()

    pltpu.sync_copy(edges_hbm.at[0, w], idx_v)

    def body(j, _):
        pltpu.sync_copy(ones_v, acc_sh.at[idx_v.at[j]], add=True)
        return 0
    lax.fori_loop(0, NCH, body, 0)

    plsc.subcore_barrier()

    def wloop(t, _):
        pltpu.sync_copy(acc_sh.at[pl.ds(s * 640 + t * 8, 8)], zb_v)
        pltpu.sync_copy(zb_v, out_hbm.at[c, pl.ds(s * 640 + t * 8, 8)])
        return 0
    lax.fori_loop(0, 80, wloop, 0)


@functools.partial(
    pl.kernel,
    out_type=jax.ShapeDtypeStruct((NC, NPAD, F), jnp.float32),
    mesh=_mesh,
    scratch_types=[
        pltpu.VMEM((NCH, CH), jnp.int32),
        pltpu.VMEM((NCH, CH), jnp.int32),
        pltpu.VMEM((CH, F), jnp.float32),
        pltpu.VMEM((8, F), jnp.float32),
        pltpu.VMEM_SHARED((NPAD, F), jnp.float32),
        pltpu.SemaphoreType.DMA,
    ],
)
def _scat_kernel(edges_hbm, hs_hbm, out_hbm, ridx_v, cidx_v, rows_v, zb_v,
                 acc_sh, sem):
    c = lax.axis_index("c")
    s = lax.axis_index("s")
    w = s * NC + c

    def fill(q, _):
        zb_v[q // 8, pl.ds((q % 8) * 16, 16)] = jnp.zeros((16,), jnp.float32)
        return 0
    lax.fori_loop(0, 64, fill, 0)

    def zloop(t, _):
        pltpu.sync_copy(zb_v, acc_sh.at[pl.ds(s * 640 + t * 8, 8)])
        return 0
    lax.fori_loop(0, 80, zloop, 0)
    plsc.subcore_barrier()

    pltpu.sync_copy(edges_hbm.at[0, w], ridx_v)
    pltpu.sync_copy(edges_hbm.at[1, w], cidx_v)

    def body(j, _):
        pltpu.async_copy(hs_hbm.at[ridx_v.at[j]], rows_v, sem).wait()
        pltpu.sync_copy(rows_v, acc_sh.at[cidx_v.at[j]], add=True)
        return 0
    lax.fori_loop(0, NCH, body, 0)

    plsc.subcore_barrier()

    def wloop(t, _):
        pltpu.sync_copy(acc_sh.at[pl.ds(s * 640 + t * 8, 8)], zb_v)
        pltpu.sync_copy(zb_v, out_hbm.at[c, pl.ds(s * 640 + t * 8, 8)])
        return 0
    lax.fori_loop(0, 80, wloop, 0)


_BR = 1000
_NB = N // _BR


def _tc1_body(x_ref, w_ref, b_ref, h_ref):
    h_ref[...] = jnp.maximum(
        jnp.dot(x_ref[...], w_ref[...], preferred_element_type=jnp.float32)
        + b_ref[...], 0.0)


def _tc2_body(deg_ref, h_ref, dis_ref, hs_ref):
    d = deg_ref[...]
    dis = jnp.where(d > 0.0, lax.rsqrt(jnp.where(d > 0.0, d, 1.0)), 0.0)
    dis_ref[...] = dis
    hs_ref[...] = dis * h_ref[...]


def _tc3_body(h_ref, s2_ref, dis_ref, bat_ref, w0_ref, w1_ref, b_ref,
              out_ref, pool_ref):
    i = pl.program_id(0)
    ssum = s2_ref[0] + s2_ref[1]
    tx1 = -(dis_ref[...] * ssum)
    o = jnp.maximum(
        jnp.dot(h_ref[...], w0_ref[...], preferred_element_type=jnp.float32)
        + jnp.dot(tx1, w1_ref[...], preferred_element_type=jnp.float32)
        + b_ref[...], 0.0)
    out_ref[...] = o
    gids = lax.broadcasted_iota(jnp.int32, (_BR, G), 1).astype(jnp.float32)
    oh = jnp.where(bat_ref[...] == gids, 1.0, 0.0)
    p = lax.dot_general(oh, o, (((0,), (0,)), ((), ())),
                        preferred_element_type=jnp.float32)

    @pl.when(i == 0)
    def _():
        pool_ref[...] = p

    @pl.when(i > 0)
    def _():
        pool_ref[...] = pool_ref[...] + p


def kernel(x2, edge_index2, batch, lin0_W, lin0_b, cheb_W0, cheb_W1, cheb_b):
    edges_r = edge_index2.astype(jnp.int32).reshape(2, NW, NCH, CH)

    deg2 = _deg_kernel(edges_r)

    h = pl.pallas_call(
        _tc1_body,
        grid=(_NB,),
        in_specs=[
            pl.BlockSpec((_BR, F), lambda i: (i, 0)),
            pl.BlockSpec((F, F), lambda i: (0, 0)),
            pl.BlockSpec((1, F), lambda i: (0, 0)),
        ],
        out_specs=pl.BlockSpec((_BR, F), lambda i: (i, 0)),
        out_shape=jax.ShapeDtypeStruct((N, F), jnp.float32),
    )(x2, lin0_W, lin0_b.reshape(1, F))

    deg_col = (deg2[0, :N, 0] + deg2[1, :N, 0]).reshape(N, 1)

    dis_col, hs = pl.pallas_call(
        _tc2_body,
        grid=(_NB,),
        in_specs=[
            pl.BlockSpec((_BR, 1), lambda i: (i, 0)),
            pl.BlockSpec((_BR, F), lambda i: (i, 0)),
        ],
        out_specs=[
            pl.BlockSpec((_BR, 1), lambda i: (i, 0)),
            pl.BlockSpec((_BR, F), lambda i: (i, 0)),
        ],
        out_shape=[
            jax.ShapeDtypeStruct((N, 1), jnp.float32),
            jax.ShapeDtypeStruct((N, F), jnp.float32),
        ],
    )(deg_col, h)

    s2 = _scat_kernel(edges_r, hs)

    bat_col = batch.astype(jnp.float32).reshape(N, 1)

    out, pooled = pl.pallas_call(
        _tc3_body,
        grid=(_NB,),
        in_specs=[
            pl.BlockSpec((_BR, F), lambda i: (i, 0)),
            pl.BlockSpec((2, _BR, F), lambda i: (0, i, 0)),
            pl.BlockSpec((_BR, 1), lambda i: (i, 0)),
            pl.BlockSpec((_BR, 1), lambda i: (i, 0)),
            pl.BlockSpec((F, F), lambda i: (0, 0)),
            pl.BlockSpec((F, F), lambda i: (0, 0)),
            pl.BlockSpec((1, F), lambda i: (0, 0)),
        ],
        out_specs=[
            pl.BlockSpec((_BR, F), lambda i: (i, 0)),
            pl.BlockSpec((G, F), lambda i: (0, 0)),
        ],
        out_shape=[
            jax.ShapeDtypeStruct((N, F), jnp.float32),
            jax.ShapeDtypeStruct((G, F), jnp.float32),
        ],
    )(h, s2, dis_col, bat_col, cheb_W0, cheb_W1, cheb_b.reshape(1, F))

    return (pooled, out)


# speedup vs baseline: 25.1424x; 1.2552x over previous
import functools
import jax
import jax.numpy as jnp
from jax import lax
from jax.experimental import pallas as pl
from jax.experimental.pallas import tpu as pltpu
from jax.experimental.pallas import tpu_sc as plsc

N = 10000
E = 320000
F = 128
G = 64

NC = 2
NS = 16
NW = NC * NS
EPT = E // NW
CH = 80
NCH = EPT // CH
NPAD = 10240
LW = 16

_mesh = plsc.VectorSubcoreMesh(core_axis_name="c", subcore_axis_name="s")


@functools.partial(
    pl.kernel,
    out_type=jax.ShapeDtypeStruct((NC, NPAD, F), jnp.float32),
    mesh=_mesh,
    scratch_types=[
        pltpu.VMEM((NCH, CH), jnp.int32),
        pltpu.VMEM((CH, F), jnp.float32),
        pltpu.VMEM((8, F), jnp.float32),
        pltpu.VMEM_SHARED((NPAD, F), jnp.float32),
        pltpu.SemaphoreType.DMA,
        pltpu.SemaphoreType.DMA,
    ],
)
def _deg_kernel(rows_hbm, out_hbm, idx_v, ones_v, zb_v, acc_sh, sem0, sem1):
    c = lax.axis_index("c")
    s = lax.axis_index("s")
    w = s * NC + c

    def fill(q, _):
        zb_v[q // 8, pl.ds((q % 8) * 16, 16)] = jnp.zeros((16,), jnp.float32)
        return 0
    lax.fori_loop(0, 64, fill, 0)

    def fill1(q, _):
        ones_v[q // 8, pl.ds((q % 8) * 16, 16)] = jnp.ones((16,), jnp.float32)
        return 0
    lax.fori_loop(0, CH * 8, fill1, 0)

    def zloop(t, _):
        pltpu.sync_copy(zb_v, acc_sh.at[pl.ds(s * 640 + t * 8, 8)])
        return 0
    lax.fori_loop(0, 80, zloop, 0)
    plsc.subcore_barrier()

    pltpu.sync_copy(rows_hbm.at[w], idx_v)

    pltpu.async_copy(ones_v, acc_sh.at[idx_v.at[0]], sem0, add=True)
    pltpu.async_copy(ones_v, acc_sh.at[idx_v.at[1]], sem1, add=True)

    def body(jj, _):
        j0 = 2 * jj
        pltpu.make_async_copy(ones_v, acc_sh.at[idx_v.at[j0]], sem0).wait()
        pltpu.async_copy(ones_v, acc_sh.at[idx_v.at[j0 + 2]], sem0, add=True)
        pltpu.make_async_copy(ones_v, acc_sh.at[idx_v.at[j0 + 1]], sem1).wait()
        pltpu.async_copy(ones_v, acc_sh.at[idx_v.at[j0 + 3]], sem1, add=True)
        return 0
    lax.fori_loop(0, (NCH - 3) // 2, body, 0)
    pltpu.make_async_copy(ones_v, acc_sh.at[idx_v.at[NCH - 3]], sem0).wait()
    pltpu.async_copy(ones_v, acc_sh.at[idx_v.at[NCH - 1]], sem0, add=True)
    pltpu.make_async_copy(ones_v, acc_sh.at[idx_v.at[NCH - 2]], sem1).wait()
    pltpu.make_async_copy(ones_v, acc_sh.at[idx_v.at[NCH - 1]], sem0).wait()

    plsc.subcore_barrier()

    def wloop(t, _):
        pltpu.sync_copy(acc_sh.at[pl.ds(s * 640 + t * 8, 8)], zb_v)
        pltpu.sync_copy(zb_v, out_hbm.at[c, pl.ds(s * 640 + t * 8, 8)])
        return 0
    lax.fori_loop(0, 80, wloop, 0)


@functools.partial(
    pl.kernel,
    out_type=jax.ShapeDtypeStruct((NC, NPAD, F), jnp.float32),
    mesh=_mesh,
    scratch_types=[
        pltpu.VMEM((NCH, CH), jnp.int32),
        pltpu.VMEM((2, 1, CH), jnp.int32),
        pltpu.VMEM((2, CH, F), jnp.float32),
        pltpu.VMEM((8, F), jnp.float32),
        pltpu.VMEM_SHARED((NPAD, F), jnp.float32),
        pltpu.SemaphoreType.DMA,
        pltpu.SemaphoreType.DMA,
        pltpu.SemaphoreType.DMA,
        pltpu.SemaphoreType.DMA,
    ],
)
def _scat_kernel(rows_hbm, cols_hbm, hs_hbm, out_hbm, ridx_v, cidx_v, rows_v,
                 zb_v, acc_sh, semg0, semg1, semc0, semc1):
    c = lax.axis_index("c")
    s = lax.axis_index("s")
    w = s * NC + c

    def fill(q, _):
        zb_v[q // 8, pl.ds((q % 8) * 16, 16)] = jnp.zeros((16,), jnp.float32)
        return 0
    lax.fori_loop(0, 64, fill, 0)

    def zloop(t, _):
        pltpu.sync_copy(zb_v, acc_sh.at[pl.ds(s * 640 + t * 8, 8)])
        return 0
    lax.fori_loop(0, 80, zloop, 0)
    plsc.subcore_barrier()

    pltpu.sync_copy(rows_hbm.at[w], ridx_v)

    def g(j, p, sem):
        return pltpu.async_copy(hs_hbm.at[ridx_v.at[j]], rows_v.at[p], sem)

    def gwait(j, p, sem):
        pltpu.make_async_copy(hs_hbm.at[ridx_v.at[j]], rows_v.at[p],
                              sem).wait()

    def cld(j, p, sem):
        return pltpu.async_copy(cols_hbm.at[w, j], cidx_v.at[p], sem)

    def cwait(j, p, sem):
        pltpu.make_async_copy(cols_hbm.at[w, j], cidx_v.at[p], sem).wait()

    def scat(p):
        pltpu.sync_copy(rows_v.at[p], acc_sh.at[cidx_v.at[p, 0]], add=True)

    cld(0, 0, semc0)
    cld(1, 1, semc1)
    g(0, 0, semg0)
    g(1, 1, semg1)

    def body(jj, _):
        j0 = 2 * jj
        gwait(j0, 0, semg0)
        cwait(j0, 0, semc0)
        scat(0)
        cld(j0 + 2, 0, semc0)
        g(j0 + 2, 0, semg0)
        gwait(j0 + 1, 1, semg1)
        cwait(j0 + 1, 1, semc1)
        scat(1)
        cld(j0 + 3, 1, semc1)
        g(j0 + 3, 1, semg1)
        return 0
    lax.fori_loop(0, (NCH - 3) // 2, body, 0)

    gwait(NCH - 3, 0, semg0)
    cwait(NCH - 3, 0, semc0)
    scat(0)
    cld(NCH - 1, 0, semc0)
    g(NCH - 1, 0, semg0)
    gwait(NCH - 2, 1, semg1)
    cwait(NCH - 2, 1, semc1)
    scat(1)
    gwait(NCH - 1, 0, semg0)
    cwait(NCH - 1, 0, semc0)
    scat(0)

    plsc.subcore_barrier()

    def wloop(t, _):
        pltpu.sync_copy(acc_sh.at[pl.ds(s * 640 + t * 8, 8)], zb_v)
        pltpu.sync_copy(zb_v, out_hbm.at[c, pl.ds(s * 640 + t * 8, 8)])
        return 0
    lax.fori_loop(0, 80, wloop, 0)


_BR = 1000
_NB = N // _BR


def _tc1_body(x_ref, w_ref, b_ref, h_ref):
    h_ref[...] = jnp.maximum(
        jnp.dot(x_ref[...], w_ref[...], preferred_element_type=jnp.float32)
        + b_ref[...], 0.0)


def _dis_of(deg2_block):
    d = deg2_block[0] + deg2_block[1]
    return jnp.where(d > 0.0, lax.rsqrt(jnp.where(d > 0.0, d, 1.0)), 0.0)


def _tc2_body(deg2_ref, h_ref, hs_ref):
    hs_ref[...] = _dis_of(deg2_ref) * h_ref[...]


def _tc3_body(h_ref, s2_ref, deg2_ref, bat_ref, w0_ref, w1_ref, b_ref,
              out_ref, pool_ref):
    i = pl.program_id(0)
    ssum = s2_ref[0] + s2_ref[1]
    tx1 = -(_dis_of(deg2_ref) * ssum)
    o = jnp.maximum(
        jnp.dot(h_ref[...], w0_ref[...], preferred_element_type=jnp.float32)
        + jnp.dot(tx1, w1_ref[...], preferred_element_type=jnp.float32)
        + b_ref[...], 0.0)
    out_ref[...] = o
    gids = lax.broadcasted_iota(jnp.int32, (_BR, G), 1).astype(jnp.float32)
    oh = jnp.where(bat_ref[...] == gids, 1.0, 0.0)
    p = lax.dot_general(oh, o, (((0,), (0,)), ((), ())),
                        preferred_element_type=jnp.float32)

    @pl.when(i == 0)
    def _():
        pool_ref[...] = p

    @pl.when(i > 0)
    def _():
        pool_ref[...] = pool_ref[...] + p


def kernel(x2, edge_index2, batch, lin0_W, lin0_b, cheb_W0, cheb_W1, cheb_b):
    ei = edge_index2.astype(jnp.int32)
    rows_r = ei[0].reshape(NW, NCH, CH)
    cols_r = ei[1].reshape(NW, NCH, 1, CH)

    deg2 = _deg_kernel(rows_r)

    h = pl.pallas_call(
        _tc1_body,
        grid=(_NB,),
        in_specs=[
            pl.BlockSpec((_BR, F), lambda i: (i, 0)),
            pl.BlockSpec((F, F), lambda i: (0, 0)),
            pl.BlockSpec((1, F), lambda i: (0, 0)),
        ],
        out_specs=pl.BlockSpec((_BR, F), lambda i: (i, 0)),
        out_shape=jax.ShapeDtypeStruct((N, F), jnp.float32),
    )(x2, lin0_W, lin0_b.reshape(1, F))

    hs = pl.pallas_call(
        _tc2_body,
        grid=(_NB,),
        in_specs=[
            pl.BlockSpec((2, _BR, F), lambda i: (0, i, 0)),
            pl.BlockSpec((_BR, F), lambda i: (i, 0)),
        ],
        out_specs=pl.BlockSpec((_BR, F), lambda i: (i, 0)),
        out_shape=jax.ShapeDtypeStruct((N, F), jnp.float32),
    )(deg2, h)

    s2 = _scat_kernel(rows_r, cols_r, hs)

    bat_col = batch.astype(jnp.float32).reshape(N, 1)

    out, pooled = pl.pallas_call(
        _tc3_body,
        grid=(_NB,),
        in_specs=[
            pl.BlockSpec((_BR, F), lambda i: (i, 0)),
            pl.BlockSpec((2, _BR, F), lambda i: (0, i, 0)),
            pl.BlockSpec((2, _BR, F), lambda i: (0, i, 0)),
            pl.BlockSpec((_BR, 1), lambda i: (i, 0)),
            pl.BlockSpec((F, F), lambda i: (0, 0)),
            pl.BlockSpec((F, F), lambda i: (0, 0)),
            pl.BlockSpec((1, F), lambda i: (0, 0)),
        ],
        out_specs=[
            pl.BlockSpec((_BR, F), lambda i: (i, 0)),
            pl.BlockSpec((G, F), lambda i: (0, 0)),
        ],
        out_shape=[
            jax.ShapeDtypeStruct((N, F), jnp.float32),
            jax.ShapeDtypeStruct((G, F), jnp.float32),
        ],
    )(h, s2, deg2, bat_col, cheb_W0, cheb_W1, cheb_b.reshape(1, F))

    return (pooled, out)
